# baseline (speedup 1.0000x reference)
import jax
import jax.numpy as jnp
from jax import lax
from jax.experimental import pallas as pl
from jax.experimental.pallas import tpu as pltpu
from jax.experimental.pallas import tpu_sc as plsc

NUM_ENT = 1000000
NUM_REL = 1000
EMB_DIM = 64
BATCH = 16384

NC = 2
NS = 16
L = 16
NW = NC * NS

NHITS = 2 * BATCH
H_PER_W = NHITS // NW
GROUPS_A = H_PER_W // L

SLAB = 512
LAST_PAGE = (NUM_ENT - 1) // SLAB
LAST_W = NUM_ENT - LAST_PAGE * SLAB

B_PER_W = BATCH // NW
CHUNK = 128
N_CHUNKS = B_PER_W // CHUNK
GROUPS_B = CHUNK // L


def _extract_body(ehT_hbm, etT_hbm, sidx_hbm, spos2_hbm,
                  stage_hbm,
                  sidx_v, spos_v, ehs, ets, rb0, rb1, sem, ssem):
    wid = lax.axis_index("s") * NC + lax.axis_index("c")
    hbase = wid * H_PER_W

    pltpu.sync_copy(sidx_hbm.at[pl.ds(hbase, H_PER_W)], sidx_v)
    pltpu.sync_copy(spos2_hbm.at[pl.ds(wid * GROUPS_A, GROUPS_A)], spos_v)

    iota16 = lax.iota(jnp.int32, L)

    def extract_hit(e, rb, i):
        page = e // SLAB

        def load_slabs(cur_page):
            @pl.when(page != cur_page)
            def _load():
                @pl.when(page == LAST_PAGE)
                def _tail():
                    off = pl.multiple_of(page * SLAB, 128)
                    pltpu.sync_copy(ehT_hbm.at[:, pl.ds(off, LAST_W)],
                                    ehs.at[:, pl.ds(0, LAST_W)])
                    pltpu.sync_copy(etT_hbm.at[:, pl.ds(off, LAST_W)],
                                    ets.at[:, pl.ds(0, LAST_W)])

                @pl.when(page != LAST_PAGE)
                def _full():
                    off = pl.multiple_of(page * SLAB, 128)
                    pltpu.sync_copy(ehT_hbm.at[:, pl.ds(off, SLAB)], ehs)
                    pltpu.sync_copy(etT_hbm.at[:, pl.ds(off, SLAB)], ets)

            return page

        e_loc = jnp.full((L,), e - page * SLAB, jnp.int32)

        def do_gathers():
            for half, slab in ((0, ehs), (1, ets)):
                for k in range(EMB_DIM // L):
                    v = plsc.load_gather(slab, [k * L + iota16, e_loc])
                    rb[i, pl.ds(half * EMB_DIM + k * L, L)] = v

        return load_slabs, do_gathers

    def group_body_a(g, cur_page):
        sv = sidx_v[pl.ds(g * L, L)]
        for i in range(L):
            e = sv[i]
            load_slabs, do_gathers = extract_hit(e, rb0, i)
            cur_page = load_slabs(cur_page)
            _, gath0 = extract_hit(e, rb0, i)
            _, gath1 = extract_hit(e, rb1, i)

            @pl.when(g % 2 == 0)
            def _even():
                gath0()

            @pl.when(g % 2 == 1)
            def _odd():
                gath1()

        @pl.when(g > 0)
        def _drain_prev():
            pltpu.make_async_copy(
                rb1, stage_hbm.at[spos_v.at[0]], ssem).wait()

        @pl.when(g % 2 == 0)
        def _f0():
            pltpu.make_async_copy(
                rb0, stage_hbm.at[spos_v.at[g]], ssem).start()

        @pl.when(g % 2 == 1)
        def _f1():
            pltpu.make_async_copy(
                rb1, stage_hbm.at[spos_v.at[g]], ssem).start()

        return cur_page

    lax.fori_loop(0, GROUPS_A, group_body_a, jnp.int32(-1), unroll=1)
    pltpu.make_async_copy(
        rb1, stage_hbm.at[spos_v.at[0]], ssem).wait()


def _score_body(heads_unused, stage_hbm, rels_hbm, relcat_hbm,
                out_hbm,
                ridx, hrow_v, trow_v, rrow_v,
                tile16, out_v, sem):
    wid = lax.axis_index("s") * NC + lax.axis_index("c")
    base = wid * B_PER_W

    iota16 = lax.iota(jnp.int32, L)

    def chunk_body(c, _):
        cbase = base + c * CHUNK
        pltpu.sync_copy(rels_hbm.at[pl.ds(cbase, CHUNK)], ridx)
        cp1 = pltpu.make_async_copy(
            stage_hbm.at[pl.ds(cbase, CHUNK), :], hrow_v, sem)
        cp2 = pltpu.make_async_copy(
            stage_hbm.at[pl.ds(BATCH + cbase, CHUNK), :], trow_v, sem)
        cp3 = pltpu.make_async_copy(relcat_hbm.at[ridx], rrow_v, sem)
        for cp in (cp1, cp2, cp3):
            cp.start()
        for cp in (cp1, cp2, cp3):
            cp.wait()

        def group_body(g, _):
            eb = g * L
            for i in range(L):
                e = eb + i
                s = None
                for k in range(EMB_DIM // L):
                    lo = pl.ds(k * L, L)
                    hi = pl.ds(EMB_DIM + k * L, L)
                    p = (hrow_v[e, lo] * rrow_v[e, lo] * trow_v[e, hi]
                         + trow_v[e, lo] * rrow_v[e, hi] * hrow_v[e, hi])
                    s = p if s is None else s + p
                tile16[i, :] = s
            acc = jnp.zeros((L,), jnp.float32)
            for j in range(L):
                col = plsc.load_gather(
                    tile16, [iota16, jnp.full((L,), j, jnp.int32)])
                acc = acc + col
            score = jnp.clip(acc * 0.5, -20.0, 20.0)
            out_v[pl.ds(c * CHUNK + eb, L)] = score
            return ()

        lax.fori_loop(0, GROUPS_B, group_body, (), unroll=1)
        return ()

    lax.fori_loop(0, N_CHUNKS, chunk_body, (), unroll=1)
    pltpu.sync_copy(out_v, out_hbm.at[pl.ds(base, B_PER_W)])


@jax.jit
def kernel(heads, rels, tails, ent_h_embs, ent_t_embs, rel_embs,
           rel_inv_embs):
    heads = heads.astype(jnp.int32)
    rels = rels.astype(jnp.int32)
    tails = tails.astype(jnp.int32)

    allidx = jnp.concatenate([heads, tails])
    slots = lax.iota(jnp.int32, NHITS)

    keys, vals = allidx, slots
    n = NHITS
    k = 2
    while k <= n:
        j = k // 2
        while j >= 1:
            kb = keys.reshape(n // (2 * j), 2, j)
            vb = vals.reshape(n // (2 * j), 2, j)
            blk = lax.iota(jnp.int32, n // (2 * j)) * (2 * j)
            asc = ((blk & k) == 0)[:, None]
            a_k, b_k = kb[:, 0, :], kb[:, 1, :]
            a_v, b_v = vb[:, 0, :], vb[:, 1, :]
            swap = jnp.where(asc, a_k > b_k, a_k < b_k)
            lo_k = jnp.where(swap, b_k, a_k)
            hi_k = jnp.where(swap, a_k, b_k)
            lo_v = jnp.where(swap, b_v, a_v)
            hi_v = jnp.where(swap, a_v, b_v)
            keys = jnp.stack([lo_k, hi_k], axis=1).reshape(n)
            vals = jnp.stack([lo_v, hi_v], axis=1).reshape(n)
            j //= 2
        k *= 2
    sidx, spos = keys, vals
    spos2 = spos.reshape(NHITS // L, L)

    relcat = jnp.concatenate([rel_embs, rel_inv_embs], axis=1)

    mesh = plsc.VectorSubcoreMesh(core_axis_name="c", subcore_axis_name="s",
                                  num_cores=NC, num_subcores=NS)
    cparams = pltpu.CompilerParams(needs_layout_passes=False,
                                   use_tc_tiling_on_sc=False)

    extract = pl.kernel(
        _extract_body,
        out_type=jax.ShapeDtypeStruct((NHITS, 2 * EMB_DIM), jnp.float32),
        mesh=mesh,
        compiler_params=cparams,
        scratch_types=[
            pltpu.VMEM((H_PER_W,), jnp.int32),
            pltpu.VMEM((GROUPS_A, L), jnp.int32),
            pltpu.VMEM((EMB_DIM, SLAB), jnp.float32),
            pltpu.VMEM((EMB_DIM, SLAB), jnp.float32),
            pltpu.VMEM((L, 2 * EMB_DIM), jnp.float32),
            pltpu.VMEM((L, 2 * EMB_DIM), jnp.float32),
            pltpu.SemaphoreType.DMA,
            pltpu.SemaphoreType.DMA,
        ],
    )
    stage = extract(ent_h_embs.T, ent_t_embs.T, sidx, spos2)

    score = pl.kernel(
        _score_body,
        out_type=jax.ShapeDtypeStruct((BATCH,), jnp.float32),
        mesh=mesh,
        compiler_params=cparams,
        scratch_types=[
            pltpu.VMEM((CHUNK,), jnp.int32),
            pltpu.VMEM((CHUNK, 2 * EMB_DIM), jnp.float32),
            pltpu.VMEM((CHUNK, 2 * EMB_DIM), jnp.float32),
            pltpu.VMEM((CHUNK, 2 * EMB_DIM), jnp.float32),
            pltpu.VMEM((L, L), jnp.float32),
            pltpu.VMEM((B_PER_W,), jnp.float32),
            pltpu.SemaphoreType.DMA,
        ],
    )
    return score(heads, stage, rels, relcat)


# speedup vs baseline: 14.3716x; 14.3716x over previous
import jax
import jax.numpy as jnp
from jax import lax
from jax.experimental import pallas as pl
from jax.experimental.pallas import tpu as pltpu
from jax.experimental.pallas import tpu_sc as plsc

NUM_ENT = 1000000
NUM_REL = 1000
EMB_DIM = 64
BATCH = 16384

NC = 2
NS = 16
L = 16
NW = NC * NS

B_PER_W = BATCH // NW
CHUNK = 128
N_CHUNKS = B_PER_W // CHUNK
GROUPS = CHUNK // L
NSEG = EMB_DIM // L


def _body(heads_hbm, rels_hbm, tails_hbm, comb_hbm, relcat_hbm,
          out_hbm,
          hidx, ridx, tidx,
          hrow_v, trow_v, rrow_v,
          tile16, out_v, sem):
    wid = lax.axis_index("s") * NC + lax.axis_index("c")
    base = wid * B_PER_W

    iota16 = lax.iota(jnp.int32, L)

    def chunk_body(c, _):
        cbase = base + c * CHUNK
        pltpu.sync_copy(heads_hbm.at[pl.ds(cbase, CHUNK)], hidx)
        pltpu.sync_copy(rels_hbm.at[pl.ds(cbase, CHUNK)], ridx)
        pltpu.sync_copy(tails_hbm.at[pl.ds(cbase, CHUNK)], tidx)
        cp1 = pltpu.make_async_copy(comb_hbm.at[hidx], hrow_v, sem)
        cp2 = pltpu.make_async_copy(comb_hbm.at[tidx], trow_v, sem)
        cp3 = pltpu.make_async_copy(relcat_hbm.at[ridx], rrow_v, sem)
        for cp in (cp1, cp2, cp3):
            cp.start()
        for cp in (cp1, cp2, cp3):
            cp.wait()

        def group_body(g, _):
            eb = g * L
            for i in range(L):
                e = eb + i
                s = None
                for k in range(NSEG):
                    lo = pl.ds(k * L, L)
                    hi = pl.ds(EMB_DIM + k * L, L)
                    p = (hrow_v[e, lo] * rrow_v[e, lo] * trow_v[e, hi]
                         + trow_v[e, lo] * rrow_v[e, hi] * hrow_v[e, hi])
                    s = p if s is None else s + p
                tile16[i, :] = s
            acc = jnp.zeros((L,), jnp.float32)
            for j in range(L):
                col = plsc.load_gather(
                    tile16, [iota16, jnp.full((L,), j, jnp.int32)])
                acc = acc + col
            score = jnp.clip(acc * 0.5, -20.0, 20.0)
            out_v[pl.ds(c * CHUNK + eb, L)] = score
            return ()

        lax.fori_loop(0, GROUPS, group_body, (), unroll=1)
        return ()

    lax.fori_loop(0, N_CHUNKS, chunk_body, (), unroll=1)
    pltpu.sync_copy(out_v, out_hbm.at[pl.ds(base, B_PER_W)])


@jax.jit
def kernel(heads, rels, tails, ent_h_embs, ent_t_embs, rel_embs,
           rel_inv_embs):
    comb = jnp.concatenate([ent_h_embs, ent_t_embs], axis=1)
    relcat = jnp.concatenate([rel_embs, rel_inv_embs], axis=1)
    mesh = plsc.VectorSubcoreMesh(core_axis_name="c", subcore_axis_name="s",
                                  num_cores=NC, num_subcores=NS)
    f = pl.kernel(
        _body,
        out_type=jax.ShapeDtypeStruct((BATCH,), jnp.float32),
        mesh=mesh,
        compiler_params=pltpu.CompilerParams(needs_layout_passes=False,
                                             use_tc_tiling_on_sc=False),
        scratch_types=[
            pltpu.VMEM((CHUNK,), jnp.int32),
            pltpu.VMEM((CHUNK,), jnp.int32),
            pltpu.VMEM((CHUNK,), jnp.int32),
            pltpu.VMEM((CHUNK, 2 * EMB_DIM), jnp.float32),
            pltpu.VMEM((CHUNK, 2 * EMB_DIM), jnp.float32),
            pltpu.VMEM((CHUNK, 2 * EMB_DIM), jnp.float32),
            pltpu.VMEM((L, L), jnp.float32),
            pltpu.VMEM((B_PER_W,), jnp.float32),
            pltpu.SemaphoreType.DMA,
        ],
    )
    return f(heads.astype(jnp.int32), rels.astype(jnp.int32),
             tails.astype(jnp.int32), comb, relcat)
